# baseline (speedup 1.0000x reference)
import functools
import math

import jax
import jax.numpy as jnp
from jax import lax
from jax.experimental import pallas as pl
from jax.experimental.pallas import tpu as pltpu
from jax.experimental.pallas import tpu_sc as plsc

EMB = 64
SCALE = math.sqrt(EMB)

_info = plsc.get_sparse_core_info()
NC = _info.num_cores
NS = _info.num_subcores
L = _info.num_lanes
NW = NC * NS

B = 4096 * 200
B_PER_W = B // NW
C = 128
N_CHUNKS = B_PER_W // C

_mesh = plsc.VectorSubcoreMesh(core_axis_name="c", subcore_axis_name="s")


@functools.partial(
    pl.kernel,
    out_type=jax.ShapeDtypeStruct((B, EMB), jnp.float32),
    mesh=_mesh,
    compiler_params=pltpu.CompilerParams(use_tc_tiling_on_sc=False),
    scratch_types=[
        pltpu.VMEM((N_CHUNKS, C), jnp.int32),
        pltpu.VMEM((C, EMB), jnp.float32),
        pltpu.VMEM((C, EMB), jnp.float32),
        pltpu.SemaphoreType.DMA,
        pltpu.SemaphoreType.DMA,
    ],
)
def _emb_kernel(tokens_hbm, table_hbm, out_hbm, idx_v, rows0, rows1, sem0, sem1):
    wid = lax.axis_index("s") * NC + lax.axis_index("c")
    base = wid * B_PER_W

    pltpu.sync_copy(tokens_hbm.at[pl.ds(wid * N_CHUNKS, N_CHUNKS)], idx_v)

    rows = (rows0, rows1)
    sems = (sem0, sem1)

    def gather_start(chunk, b):
        pltpu.async_copy(table_hbm.at[idx_v.at[chunk]], rows[b], sems[b])

    def gather_wait(chunk, b):
        pltpu.make_async_copy(
            table_hbm.at[idx_v.at[chunk]], rows[b], sems[b]).wait()

    def scale_store(chunk, b):
        def scale_row(r, _):
            for j in range(EMB // L):
                sl = pl.ds(j * L, L)
                rows[b][r, sl] = rows[b][r, sl] * SCALE
            return 0

        lax.fori_loop(0, C, scale_row, 0, unroll=4)
        pltpu.sync_copy(rows[b], out_hbm.at[pl.ds(base + chunk * C, C)])

    gather_start(0, 0)
    gather_start(1, 1)

    def body(g, _):
        i0 = g * 2

        gather_wait(i0, 0)
        scale_store(i0, 0)

        @pl.when(i0 + 2 < N_CHUNKS)
        def _():
            gather_start(i0 + 2, 0)

        gather_wait(i0 + 1, 1)
        scale_store(i0 + 1, 1)

        @pl.when(i0 + 3 < N_CHUNKS)
        def _():
            gather_start(i0 + 3, 1)

        return 0

    lax.fori_loop(0, N_CHUNKS // 2, body, 0)


def kernel(tokens, table):
    toks = tokens.reshape(B // C, C).astype(jnp.int32)
    out = _emb_kernel(toks, table)
    return out.reshape(tokens.shape + (EMB,))


# speedup vs baseline: 1.0375x; 1.0375x over previous
import functools
import math

import jax
import jax.numpy as jnp
from jax import lax
from jax.experimental import pallas as pl
from jax.experimental.pallas import tpu as pltpu
from jax.experimental.pallas import tpu_sc as plsc

EMB = 64
SCALE = math.sqrt(EMB)

_info = plsc.get_sparse_core_info()
NC = _info.num_cores
NS = _info.num_subcores
L = _info.num_lanes
NW = NC * NS

B = 4096 * 200
B_PER_W = B // NW
C = 128
N_CHUNKS = B_PER_W // C

_mesh = plsc.VectorSubcoreMesh(core_axis_name="c", subcore_axis_name="s")


NBUF = 4


@functools.partial(
    pl.kernel,
    out_type=jax.ShapeDtypeStruct((B, EMB), jnp.float32),
    mesh=_mesh,
    compiler_params=pltpu.CompilerParams(use_tc_tiling_on_sc=False),
    scratch_types=[
        pltpu.VMEM((N_CHUNKS, C), jnp.int32),
    ] + [pltpu.VMEM((C, EMB), jnp.float32) for _ in range(NBUF)]
      + [pltpu.SemaphoreType.DMA for _ in range(2 * NBUF)],
)
def _emb_kernel(tokens_hbm, table_hbm, out_hbm, idx_v, *bufs_and_sems):
    rows = bufs_and_sems[:NBUF]
    gsem = bufs_and_sems[NBUF:2 * NBUF]
    ssem = bufs_and_sems[2 * NBUF:]

    wid = lax.axis_index("s") * NC + lax.axis_index("c")
    base = wid * B_PER_W

    pltpu.sync_copy(tokens_hbm.at[pl.ds(wid * N_CHUNKS, N_CHUNKS)], idx_v)

    def gather_start(chunk, b):
        pltpu.async_copy(table_hbm.at[idx_v.at[chunk]], rows[b], gsem[b])

    def gather_wait(chunk, b):
        pltpu.make_async_copy(
            table_hbm.at[idx_v.at[chunk]], rows[b], gsem[b]).wait()

    def store_start(chunk, b):
        pltpu.async_copy(
            rows[b], out_hbm.at[pl.ds(base + chunk * C, C)], ssem[b])

    def store_wait(chunk, b):
        pltpu.make_async_copy(
            rows[b], out_hbm.at[pl.ds(base + chunk * C, C)], ssem[b]).wait()

    def scale(b):
        @plsc.parallel_loop(0, C, unroll=8)
        def _(r):
            for j in range(EMB // L):
                sl = pl.ds(j * L, L)
                rows[b][r, sl] = rows[b][r, sl] * SCALE

    for b in range(NBUF - 1):
        gather_start(b, b)

    def body(g, _):
        for b in range(NBUF):
            c = g * NBUF + b

            gather_wait(c, b)
            scale(b)
            store_start(c, b)

            nb = (b + NBUF - 1) % NBUF

            @pl.when(c + NBUF - 1 < N_CHUNKS)
            def _():
                @pl.when(c >= 1)
                def _():
                    store_wait(c - 1, nb)

                gather_start(c + NBUF - 1, nb)

        return 0

    lax.fori_loop(0, N_CHUNKS // NBUF, body, 0)


def kernel(tokens, table):
    toks = tokens.reshape(B // C, C).astype(jnp.int32)
    out = _emb_kernel(toks, table)
    return out.reshape(tokens.shape + (EMB,))
